# Initial kernel scaffold; baseline (speedup 1.0000x reference)
import jax
import jax.numpy as jnp
from jax.experimental import pallas as pl


def kernel(x, edge_index, W1, b1, W2, b2, Wl, bl):
    raise NotImplementedError("write your pallas kernel here")


# speedup vs baseline: 46.1397x; 46.1397x over previous
import functools

import jax
import jax.numpy as jnp
from jax import lax
from jax.experimental import pallas as pl
from jax.experimental.pallas import tpu as pltpu
from jax.experimental.pallas import tpu_sc as plsc

N_NODES = 10000
N_EDGES = 320000
IN_CH = 128
HID = 16

NC = 2
NS = 16
NW = NC * NS
NPAD = 10240
E_PER_W = N_EDGES // NW
CHUNK = 80
NCHUNK = E_PER_W // CHUNK
STRIPE = NPAD // NS

_MESH = plsc.VectorSubcoreMesh(core_axis_name="c", subcore_axis_name="s")


@functools.partial(
    pl.kernel,
    out_type=jax.ShapeDtypeStruct((NC, NPAD), jnp.float32),
    mesh=_MESH,
    scratch_types=[
        pltpu.VMEM((NCHUNK, CHUNK), jnp.int32),
        pltpu.VMEM((CHUNK,), jnp.float32),
        pltpu.VMEM((STRIPE,), jnp.float32),
        pltpu.VMEM_SHARED((NPAD,), jnp.float32),
    ],
)
def _deg_kernel(dst_hbm, out_hbm, dst_v, ones_v, zero_v, hist_sh):
    c = lax.axis_index("c")
    s = lax.axis_index("s")
    wid = c * NS + s

    for i in range(STRIPE // 16):
        zero_v[pl.ds(i * 16, 16)] = jnp.zeros((16,), jnp.float32)
    for i in range(CHUNK // 16):
        ones_v[pl.ds(i * 16, 16)] = jnp.ones((16,), jnp.float32)

    pltpu.sync_copy(zero_v, hist_sh.at[pl.ds(s * STRIPE, STRIPE)])
    plsc.subcore_barrier()

    pltpu.sync_copy(dst_hbm.at[wid], dst_v)

    def body(j, _):
        pltpu.sync_copy(ones_v, hist_sh.at[dst_v.at[j]], add=True)
        return 0
    lax.fori_loop(0, NCHUNK, body, 0)

    plsc.subcore_barrier()
    pltpu.sync_copy(hist_sh.at[pl.ds(s * STRIPE, STRIPE)],
                    out_hbm.at[c, pl.ds(s * STRIPE, STRIPE)])


@functools.partial(
    pl.kernel,
    out_type=jax.ShapeDtypeStruct((NC, NPAD, HID), jnp.float32),
    mesh=_MESH,
    scratch_types=[
        pltpu.VMEM((NCHUNK, CHUNK), jnp.int32),
        pltpu.VMEM((NCHUNK, CHUNK), jnp.int32),
        pltpu.VMEM((CHUNK, HID), jnp.float32),
        pltpu.VMEM((CHUNK, HID), jnp.float32),
        pltpu.VMEM((STRIPE, HID), jnp.float32),
        pltpu.VMEM_SHARED((NPAD, HID), jnp.float32),
        pltpu.SemaphoreType.DMA,
        pltpu.SemaphoreType.DMA,
    ],
    compiler_params=pltpu.CompilerParams(use_tc_tiling_on_sc=False),
)
def _mp_kernel(xs_hbm, src_hbm, dst_hbm, out_hbm,
               src_v, dst_v, rows0_v, rows1_v, zero_v, acc_sh, sem0, sem1):
    c = lax.axis_index("c")
    s = lax.axis_index("s")
    wid = c * NS + s

    def zbody(i, _):
        zero_v[i] = jnp.zeros((HID,), jnp.float32)
        return 0
    lax.fori_loop(0, STRIPE, zbody, 0, unroll=8)
    pltpu.sync_copy(zero_v, acc_sh.at[pl.ds(s * STRIPE, STRIPE)])
    plsc.subcore_barrier()

    pltpu.sync_copy(src_hbm.at[wid], src_v)
    pltpu.sync_copy(dst_hbm.at[wid], dst_v)

    bufs = (rows0_v, rows1_v)
    sems = (sem0, sem1)
    pltpu.async_copy(xs_hbm.at[src_v.at[0]], rows0_v, sem0)

    def body(j, _):
        for p in range(2):
            @pl.when(lax.rem(j, 2) == p)
            def _():
                cur, nxt = bufs[p], bufs[1 - p]
                csem, nsem = sems[p], sems[1 - p]

                @pl.when(j + 1 < NCHUNK)
                def _():
                    pltpu.async_copy(xs_hbm.at[src_v.at[j + 1]], nxt, nsem)

                pltpu.make_async_copy(xs_hbm.at[src_v.at[j]], cur, csem).wait()
                pltpu.sync_copy(cur, acc_sh.at[dst_v.at[j]], add=True)
        return 0
    lax.fori_loop(0, NCHUNK, body, 0)

    plsc.subcore_barrier()
    pltpu.sync_copy(acc_sh.at[pl.ds(s * STRIPE, STRIPE)],
                    out_hbm.at[c, pl.ds(s * STRIPE, STRIPE)])


def _tc_prep_body(x_ref, w1_ref, degp_ref, xs1_ref, dis_ref):
    deg = degp_ref[0, :N_NODES] + degp_ref[1, :N_NODES] + 1.0
    dis = lax.rsqrt(deg)
    dis_ref[...] = dis
    xw = jnp.dot(x_ref[...], w1_ref[...], preferred_element_type=jnp.float32)
    xs1_ref[...] = dis[:, None] * xw


def _tc_mid_body(accp_ref, xs1_ref, dis_ref, w2_ref, b1_ref, xs2_ref):
    acc = accp_ref[0, :N_NODES, :] + accp_ref[1, :N_NODES, :] + xs1_ref[...]
    dis = dis_ref[...]
    h1 = dis[:, None] * acc + b1_ref[...][None, :]
    h1 = jnp.where(h1 > 0, h1, jnp.exp(jnp.minimum(h1, 0.0)) - 1.0)
    xw2 = jnp.dot(h1, w2_ref[...], preferred_element_type=jnp.float32)
    xs2_ref[...] = dis[:, None] * xw2


def _tc_final_body(accp_ref, xs2_ref, dis_ref, wl_ref, b2_ref, bl_ref, out_ref):
    acc = accp_ref[0, :N_NODES, :] + accp_ref[1, :N_NODES, :] + xs2_ref[...]
    h2 = dis_ref[...][:, None] * acc + b2_ref[...][None, :]
    out_ref[...] = (
        jnp.dot(h2, wl_ref[...], preferred_element_type=jnp.float32)
        + bl_ref[...][None, :]
    )


def kernel(x, edge_index, W1, b1, W2, b2, Wl, bl):
    src = edge_index[0].astype(jnp.int32).reshape(NW, NCHUNK, CHUNK)
    dst = edge_index[1].astype(jnp.int32).reshape(NW, NCHUNK, CHUNK)

    degp = _deg_kernel(dst)

    xs1, dis = pl.pallas_call(
        _tc_prep_body,
        out_shape=(
            jax.ShapeDtypeStruct((N_NODES, HID), jnp.float32),
            jax.ShapeDtypeStruct((N_NODES,), jnp.float32),
        ),
    )(x, W1, degp)

    acc1 = _mp_kernel(xs1, src, dst)

    xs2 = pl.pallas_call(
        _tc_mid_body,
        out_shape=jax.ShapeDtypeStruct((N_NODES, HID), jnp.float32),
    )(acc1, xs1, dis, W2, b1)

    acc2 = _mp_kernel(xs2, src, dst)

    out = pl.pallas_call(
        _tc_final_body,
        out_shape=jax.ShapeDtypeStruct((N_NODES, 1), jnp.float32),
    )(acc2, xs2, dis, Wl, b2, bl)

    return out


# speedup vs baseline: 60.3614x; 1.3082x over previous
import functools

import jax
import jax.numpy as jnp
from jax import lax
from jax.experimental import pallas as pl
from jax.experimental.pallas import tpu as pltpu
from jax.experimental.pallas import tpu_sc as plsc

N_NODES = 10000
N_EDGES = 320000
IN_CH = 128
HID = 16

NC = 2
NS = 16
NW = NC * NS
NPAD = 10240
E_PER_W = N_EDGES // NW
CHUNK = 80
NCHUNK = E_PER_W // CHUNK
STRIPE = NPAD // NS

_MESH = plsc.VectorSubcoreMesh(core_axis_name="c", subcore_axis_name="s")


@functools.partial(
    pl.kernel,
    out_type=jax.ShapeDtypeStruct((NC, NPAD), jnp.float32),
    mesh=_MESH,
    scratch_types=[
        pltpu.VMEM((NCHUNK, CHUNK), jnp.int32),
        pltpu.VMEM((CHUNK,), jnp.float32),
        pltpu.VMEM((STRIPE,), jnp.float32),
        pltpu.VMEM_SHARED((NPAD,), jnp.float32),
        pltpu.SemaphoreType.DMA,
    ],
    compiler_params=pltpu.CompilerParams(use_tc_tiling_on_sc=False),
)
def _deg_kernel(dst_hbm, out_hbm, dst_v, ones_v, zero_v, hist_sh, sem):
    c = lax.axis_index("c")
    s = lax.axis_index("s")
    wid = c * NS + s

    for i in range(STRIPE // 16):
        zero_v[pl.ds(i * 16, 16)] = jnp.zeros((16,), jnp.float32)
    for i in range(CHUNK // 16):
        ones_v[pl.ds(i * 16, 16)] = jnp.ones((16,), jnp.float32)

    pltpu.sync_copy(zero_v, hist_sh.at[pl.ds(s * STRIPE, STRIPE)])
    plsc.subcore_barrier()

    pltpu.sync_copy(dst_hbm.at[wid], dst_v)

    def body(j, _):
        pltpu.async_copy(ones_v, hist_sh.at[dst_v.at[j]], sem, add=True)
        return 0
    lax.fori_loop(0, NCHUNK, body, 0)

    def drain(j, _):
        pltpu.make_async_copy(ones_v, hist_sh.at[dst_v.at[0]], sem).wait()
        return 0
    lax.fori_loop(0, NCHUNK, drain, 0)

    plsc.subcore_barrier()
    pltpu.sync_copy(hist_sh.at[pl.ds(s * STRIPE, STRIPE)],
                    out_hbm.at[c, pl.ds(s * STRIPE, STRIPE)])


NSLOT = 5
NOUTER = NCHUNK // NSLOT


@functools.partial(
    pl.kernel,
    out_type=jax.ShapeDtypeStruct((NC, NPAD, HID), jnp.float32),
    mesh=_MESH,
    scratch_types=[
        pltpu.VMEM((NCHUNK, CHUNK), jnp.int32),
        pltpu.VMEM((NCHUNK, CHUNK), jnp.int32),
        [pltpu.VMEM((CHUNK, HID), jnp.float32) for _ in range(NSLOT)],
        pltpu.VMEM((STRIPE, HID), jnp.float32),
        pltpu.VMEM_SHARED((NPAD, HID), jnp.float32),
        [pltpu.SemaphoreType.DMA for _ in range(NSLOT)],
        [pltpu.SemaphoreType.DMA for _ in range(NSLOT)],
    ],
    compiler_params=pltpu.CompilerParams(use_tc_tiling_on_sc=False),
)
def _mp_kernel(xs_hbm, src_hbm, dst_hbm, out_hbm,
               src_v, dst_v, bufs, zero_v, acc_sh, gsems, ssems):
    c = lax.axis_index("c")
    s = lax.axis_index("s")
    wid = c * NS + s

    def zbody(i, _):
        zero_v[i] = jnp.zeros((HID,), jnp.float32)
        return 0
    lax.fori_loop(0, STRIPE, zbody, 0, unroll=8)
    pltpu.sync_copy(zero_v, acc_sh.at[pl.ds(s * STRIPE, STRIPE)])
    plsc.subcore_barrier()

    pltpu.sync_copy(src_hbm.at[wid], src_v)
    pltpu.sync_copy(dst_hbm.at[wid], dst_v)

    pltpu.async_copy(xs_hbm.at[src_v.at[0]], bufs[0], gsems[0])
    pltpu.async_copy(xs_hbm.at[src_v.at[1]], bufs[1], gsems[1])

    def body(t, _):
        for b in range(NSLOT):
            j = t * NSLOT + b
            p2 = (b + 2) % NSLOT

            @pl.when(j >= 3)
            def _():
                pltpu.make_async_copy(
                    bufs[p2], acc_sh.at[dst_v.at[0]], ssems[p2]).wait()

            @pl.when(j + 2 < NCHUNK)
            def _():
                pltpu.async_copy(xs_hbm.at[src_v.at[j + 2]], bufs[p2],
                                 gsems[p2])

            pltpu.make_async_copy(xs_hbm.at[src_v.at[j]], bufs[b],
                                  gsems[b]).wait()
            pltpu.async_copy(bufs[b], acc_sh.at[dst_v.at[j]], ssems[b],
                             add=True)
        return 0
    lax.fori_loop(0, NOUTER, body, 0)

    for b in (2, 3, 4):
        pltpu.make_async_copy(bufs[b], acc_sh.at[dst_v.at[0]],
                              ssems[b]).wait()

    plsc.subcore_barrier()
    pltpu.sync_copy(acc_sh.at[pl.ds(s * STRIPE, STRIPE)],
                    out_hbm.at[c, pl.ds(s * STRIPE, STRIPE)])


def _tc_prep_body(x_ref, w1_ref, degp_ref, xs1_ref, dis_ref):
    deg = degp_ref[0, :N_NODES] + degp_ref[1, :N_NODES] + 1.0
    dis = lax.rsqrt(deg)
    dis_ref[...] = dis
    xw = jnp.dot(x_ref[...], w1_ref[...], preferred_element_type=jnp.float32)
    xs1_ref[...] = dis[:, None] * xw


def _tc_mid_body(accp_ref, xs1_ref, dis_ref, w2_ref, b1_ref, xs2_ref):
    acc = accp_ref[0, :N_NODES, :] + accp_ref[1, :N_NODES, :] + xs1_ref[...]
    dis = dis_ref[...]
    h1 = dis[:, None] * acc + b1_ref[...][None, :]
    h1 = jnp.where(h1 > 0, h1, jnp.exp(jnp.minimum(h1, 0.0)) - 1.0)
    xw2 = jnp.dot(h1, w2_ref[...], preferred_element_type=jnp.float32)
    xs2_ref[...] = dis[:, None] * xw2


def _tc_final_body(accp_ref, xs2_ref, dis_ref, wl_ref, b2_ref, bl_ref, out_ref):
    acc = accp_ref[0, :N_NODES, :] + accp_ref[1, :N_NODES, :] + xs2_ref[...]
    h2 = dis_ref[...][:, None] * acc + b2_ref[...][None, :]
    out_ref[...] = (
        jnp.dot(h2, wl_ref[...], preferred_element_type=jnp.float32)
        + bl_ref[...][None, :]
    )


def kernel(x, edge_index, W1, b1, W2, b2, Wl, bl):
    src = edge_index[0].astype(jnp.int32).reshape(NW, NCHUNK, CHUNK)
    dst = edge_index[1].astype(jnp.int32).reshape(NW, NCHUNK, CHUNK)

    degp = _deg_kernel(dst)

    xs1, dis = pl.pallas_call(
        _tc_prep_body,
        out_shape=(
            jax.ShapeDtypeStruct((N_NODES, HID), jnp.float32),
            jax.ShapeDtypeStruct((N_NODES,), jnp.float32),
        ),
    )(x, W1, degp)

    acc1 = _mp_kernel(xs1, src, dst)

    xs2 = pl.pallas_call(
        _tc_mid_body,
        out_shape=jax.ShapeDtypeStruct((N_NODES, HID), jnp.float32),
    )(acc1, xs1, dis, W2, b1)

    acc2 = _mp_kernel(xs2, src, dst)

    out = pl.pallas_call(
        _tc_final_body,
        out_shape=jax.ShapeDtypeStruct((N_NODES, 1), jnp.float32),
    )(acc2, xs2, dis, Wl, b2, bl)

    return out


# speedup vs baseline: 81.5096x; 1.3504x over previous
import functools

import jax
import jax.numpy as jnp
from jax import lax
from jax.experimental import pallas as pl
from jax.experimental.pallas import tpu as pltpu
from jax.experimental.pallas import tpu_sc as plsc

N_NODES = 10000
N_EDGES = 320000
IN_CH = 128
HID = 16

NC = 2
NS = 16
NW = NC * NS
NPAD = 10240
E_PER_W = N_EDGES // NW
CHUNK = 80
NCHUNK = E_PER_W // CHUNK
STRIPE = NPAD // NS

GRP = N_NODES * HID // 128
GRP_PAD = NPAD * HID // 128
DEG_G = NPAD // 128

_MESH = plsc.VectorSubcoreMesh(core_axis_name="c", subcore_axis_name="s")


@functools.partial(
    pl.kernel,
    out_type=jax.ShapeDtypeStruct((NC, NPAD, HID), jnp.float32),
    mesh=_MESH,
    scratch_types=[
        pltpu.VMEM((E_PER_W,), jnp.int32),
        pltpu.VMEM((CHUNK, HID), jnp.float32),
        pltpu.VMEM((STRIPE, HID), jnp.float32),
        pltpu.VMEM_SHARED((NPAD, HID), jnp.float32),
        pltpu.SemaphoreType.DMA,
    ],
    compiler_params=pltpu.CompilerParams(use_tc_tiling_on_sc=False),
)
def _deg_kernel(edge_hbm, out_hbm, dst_v, ones_v, zero_v, hist_sh, sem):
    c = lax.axis_index("c")
    s = lax.axis_index("s")
    wid = c * NS + s

    def zb(i, _):
        zero_v[i] = jnp.zeros((HID,), jnp.float32)
        return 0
    lax.fori_loop(0, STRIPE, zb, 0, unroll=8)

    def ob(i, _):
        ones_v[i] = jnp.ones((HID,), jnp.float32)
        return 0
    lax.fori_loop(0, CHUNK, ob, 0, unroll=8)

    pltpu.sync_copy(zero_v, hist_sh.at[pl.ds(s * STRIPE, STRIPE)])
    plsc.subcore_barrier()

    pltpu.sync_copy(edge_hbm.at[1, pl.ds(wid * E_PER_W, E_PER_W)], dst_v)

    def body(j, _):
        pltpu.async_copy(
            ones_v, hist_sh.at[dst_v.at[pl.ds(j * CHUNK, CHUNK)]], sem,
            add=True)
        return 0
    lax.fori_loop(0, NCHUNK, body, 0)

    def drain(j, _):
        pltpu.make_async_copy(
            ones_v, hist_sh.at[dst_v.at[pl.ds(0, CHUNK)]], sem).wait()
        return 0
    lax.fori_loop(0, NCHUNK, drain, 0)

    plsc.subcore_barrier()
    pltpu.sync_copy(hist_sh.at[pl.ds(s * STRIPE, STRIPE)],
                    out_hbm.at[c, pl.ds(s * STRIPE, STRIPE)])


NSLOT = 5
NOUTER = NCHUNK // NSLOT


@functools.partial(
    pl.kernel,
    out_type=jax.ShapeDtypeStruct((NC, NPAD, HID), jnp.float32),
    mesh=_MESH,
    scratch_types=[
        pltpu.VMEM((E_PER_W,), jnp.int32),
        pltpu.VMEM((E_PER_W,), jnp.int32),
        [pltpu.VMEM((CHUNK, HID), jnp.float32) for _ in range(NSLOT)],
        pltpu.VMEM((STRIPE, HID), jnp.float32),
        pltpu.VMEM_SHARED((NPAD, HID), jnp.float32),
        [pltpu.SemaphoreType.DMA for _ in range(NSLOT)],
        [pltpu.SemaphoreType.DMA for _ in range(NSLOT)],
    ],
    compiler_params=pltpu.CompilerParams(use_tc_tiling_on_sc=False),
)
def _mp_kernel(xs_hbm, edge_hbm, out_hbm,
               src_v, dst_v, bufs, zero_v, acc_sh, gsems, ssems):
    c = lax.axis_index("c")
    s = lax.axis_index("s")
    wid = c * NS + s

    def zbody(i, _):
        zero_v[i] = jnp.zeros((HID,), jnp.float32)
        return 0
    lax.fori_loop(0, STRIPE, zbody, 0, unroll=8)
    pltpu.sync_copy(zero_v, acc_sh.at[pl.ds(s * STRIPE, STRIPE)])
    plsc.subcore_barrier()

    pltpu.sync_copy(edge_hbm.at[0, pl.ds(wid * E_PER_W, E_PER_W)], src_v)
    pltpu.sync_copy(edge_hbm.at[1, pl.ds(wid * E_PER_W, E_PER_W)], dst_v)

    def sidx(j):
        return src_v.at[pl.ds(j * CHUNK, CHUNK)]

    def didx(j):
        return dst_v.at[pl.ds(j * CHUNK, CHUNK)]

    pltpu.async_copy(xs_hbm.at[sidx(0)], bufs[0], gsems[0])
    pltpu.async_copy(xs_hbm.at[sidx(1)], bufs[1], gsems[1])

    def body(t, _):
        for b in range(NSLOT):
            j = t * NSLOT + b
            p2 = (b + 2) % NSLOT

            @pl.when(j >= 3)
            def _():
                pltpu.make_async_copy(
                    bufs[p2], acc_sh.at[didx(0)], ssems[p2]).wait()

            @pl.when(j + 2 < NCHUNK)
            def _():
                pltpu.async_copy(xs_hbm.at[sidx(j + 2)], bufs[p2],
                                 gsems[p2])

            pltpu.make_async_copy(xs_hbm.at[sidx(j)], bufs[b],
                                  gsems[b]).wait()
            pltpu.async_copy(bufs[b], acc_sh.at[didx(j)], ssems[b],
                             add=True)
        return 0
    lax.fori_loop(0, NOUTER, body, 0)

    for b in (2, 3, 4):
        pltpu.make_async_copy(bufs[b], acc_sh.at[didx(0)], ssems[b]).wait()

    plsc.subcore_barrier()
    pltpu.sync_copy(acc_sh.at[pl.ds(s * STRIPE, STRIPE)],
                    out_hbm.at[c, pl.ds(s * STRIPE, STRIPE)])


def _tc_prep_body(xr_ref, w1b_ref, degp_ref, xs1_ref, dis_ref):
    deg = degp_ref[0, :GRP] + degp_ref[1, :GRP] + 1.0
    dis_g = lax.rsqrt(deg)
    dis_ref[...] = dis_g
    xw_g = jnp.dot(xr_ref[...], w1b_ref[...],
                   preferred_element_type=jnp.float32)
    xs1_ref[...] = dis_g * xw_g


def _tc_mid_body(accp_ref, xs1_ref, dis_ref, w2b_ref, b1t_ref, xs2_ref):
    acc = accp_ref[0, :GRP] + accp_ref[1, :GRP] + xs1_ref[...]
    dis_g = dis_ref[...]
    h1 = dis_g * acc + b1t_ref[...]
    h1 = jnp.where(h1 > 0, h1, jnp.exp(jnp.minimum(h1, 0.0)) - 1.0)
    xw2 = jnp.dot(h1, w2b_ref[...], preferred_element_type=jnp.float32)
    xs2_ref[...] = dis_g * xw2


def _tc_final_body(accp_ref, xs2_ref, dis_ref, wlb_ref, b2t_ref, blr_ref,
                   out_ref):
    acc = accp_ref[0, :GRP] + accp_ref[1, :GRP] + xs2_ref[...]
    h2 = dis_ref[...] * acc + b2t_ref[...]
    og = jnp.dot(h2, wlb_ref[...], preferred_element_type=jnp.float32)
    out_ref[...] = og + blr_ref[...]


def kernel(x, edge_index, W1, b1, W2, b2, Wl, bl):
    e32 = edge_index.astype(jnp.int32)

    eye8 = jnp.eye(8, dtype=jnp.float32)
    w1b = jnp.kron(eye8, W1)
    w2b = jnp.kron(eye8, W2)
    wlb = jnp.kron(eye8, Wl)
    b1t = jnp.tile(b1, 8).reshape(1, 128)
    b2t = jnp.tile(b2, 8).reshape(1, 128)
    blr = bl.reshape(1, 1)
    xr = x.reshape(GRP, 8 * IN_CH)

    degp = _deg_kernel(e32)
    degp_g = degp.reshape(NC, GRP_PAD, 128)

    xs1_g, dis_g = pl.pallas_call(
        _tc_prep_body,
        out_shape=(
            jax.ShapeDtypeStruct((GRP, 128), jnp.float32),
            jax.ShapeDtypeStruct((GRP, 128), jnp.float32),
        ),
    )(xr, w1b, degp_g)

    acc1 = _mp_kernel(xs1_g.reshape(N_NODES, HID), e32)
    acc1_g = acc1.reshape(NC, GRP_PAD, 128)

    xs2_g = pl.pallas_call(
        _tc_mid_body,
        out_shape=jax.ShapeDtypeStruct((GRP, 128), jnp.float32),
    )(acc1_g, xs1_g, dis_g, w2b, b1t)

    acc2 = _mp_kernel(xs2_g.reshape(N_NODES, HID), e32)
    acc2_g = acc2.reshape(NC, GRP_PAD, 128)

    out_g = pl.pallas_call(
        _tc_final_body,
        out_shape=jax.ShapeDtypeStruct((GRP, 8), jnp.float32),
    )(acc2_g, xs2_g, dis_g, wlb, b2t, blr)

    return out_g.reshape(N_NODES, 1)


# speedup vs baseline: 114.6521x; 1.4066x over previous
import functools

import jax
import jax.numpy as jnp
from jax import lax
from jax.experimental import pallas as pl
from jax.experimental.pallas import tpu as pltpu
from jax.experimental.pallas import tpu_sc as plsc

N_NODES = 10000
N_EDGES = 320000
IN_CH = 128
HID = 16

NC = 2
NS = 16
NW = NC * NS
NPAD = 10240
E_PER_W = N_EDGES // NW
CHUNK = 80
NCHUNK = E_PER_W // CHUNK
STRIPE = NPAD // NS

GRP = N_NODES * HID // 128
GRP_PAD = NPAD * HID // 128
DEG_G = NPAD // 128

_MESH = plsc.VectorSubcoreMesh(core_axis_name="c", subcore_axis_name="s")


@functools.partial(
    pl.kernel,
    out_type=jax.ShapeDtypeStruct((NC, NPAD, HID), jnp.float32),
    mesh=_MESH,
    scratch_types=[
        pltpu.VMEM((E_PER_W,), jnp.int32),
        pltpu.VMEM((CHUNK, HID), jnp.float32),
        pltpu.VMEM((STRIPE, HID), jnp.float32),
        pltpu.VMEM_SHARED((NPAD, HID), jnp.float32),
        pltpu.SemaphoreType.DMA,
    ],
    compiler_params=pltpu.CompilerParams(use_tc_tiling_on_sc=False),
)
def _deg_kernel(edge_hbm, out_hbm, dst_v, ones_v, zero_v, hist_sh, sem):
    c = lax.axis_index("c")
    s = lax.axis_index("s")
    wid = c * NS + s

    def zb(i, _):
        zero_v[i] = jnp.zeros((HID,), jnp.float32)
        return 0
    lax.fori_loop(0, STRIPE, zb, 0, unroll=8)

    def ob(i, _):
        ones_v[i] = jnp.ones((HID,), jnp.float32)
        return 0
    lax.fori_loop(0, CHUNK, ob, 0, unroll=8)

    pltpu.sync_copy(zero_v, hist_sh.at[pl.ds(s * STRIPE, STRIPE)])
    plsc.subcore_barrier()

    pltpu.sync_copy(edge_hbm.at[1, pl.ds(wid * E_PER_W, E_PER_W)], dst_v)

    def body(j, _):
        pltpu.async_copy(
            ones_v, hist_sh.at[dst_v.at[pl.ds(j * CHUNK, CHUNK)]], sem,
            add=True)
        return 0
    lax.fori_loop(0, NCHUNK, body, 0)

    def drain(j, _):
        pltpu.make_async_copy(
            ones_v, hist_sh.at[dst_v.at[pl.ds(0, CHUNK)]], sem).wait()
        return 0
    lax.fori_loop(0, NCHUNK, drain, 0)

    plsc.subcore_barrier()
    pltpu.sync_copy(hist_sh.at[pl.ds(s * STRIPE, STRIPE)],
                    out_hbm.at[c, pl.ds(s * STRIPE, STRIPE)])


NSLOT = 5
NOUTER = NCHUNK // NSLOT


@functools.partial(
    pl.kernel,
    out_type=jax.ShapeDtypeStruct((NC, NPAD, HID), jnp.float32),
    mesh=_MESH,
    scratch_types=[
        pltpu.VMEM((E_PER_W,), jnp.int32),
        pltpu.VMEM((E_PER_W,), jnp.int32),
        [pltpu.VMEM((CHUNK, HID), jnp.float32) for _ in range(NSLOT)],
        pltpu.VMEM((STRIPE, HID), jnp.float32),
        pltpu.VMEM_SHARED((NPAD, HID), jnp.float32),
        pltpu.VMEM_SHARED((N_NODES, HID), jnp.float32),
        [pltpu.SemaphoreType.DMA for _ in range(NSLOT)],
        [pltpu.SemaphoreType.DMA for _ in range(NSLOT)],
    ],
    compiler_params=pltpu.CompilerParams(use_tc_tiling_on_sc=False),
)
def _mp_kernel(xs_hbm, edge_hbm, out_hbm,
               src_v, dst_v, bufs, zero_v, acc_sh, xs_sh, gsems, ssems):
    c = lax.axis_index("c")
    s = lax.axis_index("s")
    wid = c * NS + s
    XSTRIPE = N_NODES // NS

    def zbody(i, _):
        zero_v[i] = jnp.zeros((HID,), jnp.float32)
        return 0
    lax.fori_loop(0, STRIPE, zbody, 0, unroll=8)
    pltpu.sync_copy(zero_v, acc_sh.at[pl.ds(s * STRIPE, STRIPE)])
    pltpu.sync_copy(xs_hbm.at[pl.ds(s * XSTRIPE, XSTRIPE)],
                    xs_sh.at[pl.ds(s * XSTRIPE, XSTRIPE)])
    plsc.subcore_barrier()

    pltpu.sync_copy(edge_hbm.at[0, pl.ds(wid * E_PER_W, E_PER_W)], src_v)
    pltpu.sync_copy(edge_hbm.at[1, pl.ds(wid * E_PER_W, E_PER_W)], dst_v)

    def sidx(j):
        return src_v.at[pl.ds(j * CHUNK, CHUNK)]

    def didx(j):
        return dst_v.at[pl.ds(j * CHUNK, CHUNK)]

    pltpu.async_copy(xs_sh.at[sidx(0)], bufs[0], gsems[0])
    pltpu.async_copy(xs_sh.at[sidx(1)], bufs[1], gsems[1])

    def body(t, _):
        for b in range(NSLOT):
            j = t * NSLOT + b
            p2 = (b + 2) % NSLOT

            @pl.when(j >= 3)
            def _():
                pltpu.make_async_copy(
                    bufs[p2], acc_sh.at[didx(0)], ssems[p2]).wait()

            @pl.when(j + 2 < NCHUNK)
            def _():
                pltpu.async_copy(xs_sh.at[sidx(j + 2)], bufs[p2],
                                 gsems[p2])

            pltpu.make_async_copy(xs_sh.at[sidx(j)], bufs[b],
                                  gsems[b]).wait()
            pltpu.async_copy(bufs[b], acc_sh.at[didx(j)], ssems[b],
                             add=True)
        return 0
    lax.fori_loop(0, NOUTER, body, 0)

    for b in (2, 3, 4):
        pltpu.make_async_copy(bufs[b], acc_sh.at[didx(0)], ssems[b]).wait()

    plsc.subcore_barrier()
    pltpu.sync_copy(acc_sh.at[pl.ds(s * STRIPE, STRIPE)],
                    out_hbm.at[c, pl.ds(s * STRIPE, STRIPE)])


def _tc_prep_body(xr_ref, w1b_ref, degp_ref, xs1_ref, dis_ref):
    deg = degp_ref[0, :GRP] + degp_ref[1, :GRP] + 1.0
    dis_g = lax.rsqrt(deg)
    dis_ref[...] = dis_g
    xw_g = jnp.dot(xr_ref[...], w1b_ref[...],
                   preferred_element_type=jnp.float32)
    xs1_ref[...] = dis_g * xw_g


def _tc_mid_body(accp_ref, xs1_ref, dis_ref, w2b_ref, b1t_ref, xs2_ref):
    acc = accp_ref[0, :GRP] + accp_ref[1, :GRP] + xs1_ref[...]
    dis_g = dis_ref[...]
    h1 = dis_g * acc + b1t_ref[...]
    h1 = jnp.where(h1 > 0, h1, jnp.exp(jnp.minimum(h1, 0.0)) - 1.0)
    xw2 = jnp.dot(h1, w2b_ref[...], preferred_element_type=jnp.float32)
    xs2_ref[...] = dis_g * xw2


def _tc_final_body(accp_ref, xs2_ref, dis_ref, wlb_ref, b2t_ref, blr_ref,
                   out_ref):
    acc = accp_ref[0, :GRP] + accp_ref[1, :GRP] + xs2_ref[...]
    h2 = dis_ref[...] * acc + b2t_ref[...]
    og = jnp.dot(h2, wlb_ref[...], preferred_element_type=jnp.float32)
    out_ref[...] = og + blr_ref[...]


def kernel(x, edge_index, W1, b1, W2, b2, Wl, bl):
    e32 = edge_index.astype(jnp.int32)

    eye8 = jnp.eye(8, dtype=jnp.float32)
    w1b = jnp.kron(eye8, W1)
    w2b = jnp.kron(eye8, W2)
    wlb = jnp.kron(eye8, Wl)
    b1t = jnp.tile(b1, 8).reshape(1, 128)
    b2t = jnp.tile(b2, 8).reshape(1, 128)
    blr = bl.reshape(1, 1)
    xr = x.reshape(GRP, 8 * IN_CH)

    degp = _deg_kernel(e32)
    degp_g = degp.reshape(NC, GRP_PAD, 128)

    xs1_g, dis_g = pl.pallas_call(
        _tc_prep_body,
        out_shape=(
            jax.ShapeDtypeStruct((GRP, 128), jnp.float32),
            jax.ShapeDtypeStruct((GRP, 128), jnp.float32),
        ),
    )(xr, w1b, degp_g)

    acc1 = _mp_kernel(xs1_g.reshape(N_NODES, HID), e32)
    acc1_g = acc1.reshape(NC, GRP_PAD, 128)

    xs2_g = pl.pallas_call(
        _tc_mid_body,
        out_shape=jax.ShapeDtypeStruct((GRP, 128), jnp.float32),
    )(acc1_g, xs1_g, dis_g, w2b, b1t)

    acc2 = _mp_kernel(xs2_g.reshape(N_NODES, HID), e32)
    acc2_g = acc2.reshape(NC, GRP_PAD, 128)

    out_g = pl.pallas_call(
        _tc_final_body,
        out_shape=jax.ShapeDtypeStruct((GRP, 8), jnp.float32),
    )(acc2_g, xs2_g, dis_g, wlb, b2t, blr)

    return out_g.reshape(N_NODES, 1)


# speedup vs baseline: 117.0684x; 1.0211x over previous
import functools

import jax
import jax.numpy as jnp
from jax import lax
from jax.experimental import pallas as pl
from jax.experimental.pallas import tpu as pltpu
from jax.experimental.pallas import tpu_sc as plsc

N_NODES = 10000
N_EDGES = 320000
IN_CH = 128
HID = 16

NC = 2
NS = 16
NW = NC * NS
NPAD = 10240
E_PER_W = N_EDGES // NW
CHUNK = 80
NCHUNK = E_PER_W // CHUNK
STRIPE = NPAD // NS

GRP = N_NODES * HID // 128
GRP_PAD = NPAD * HID // 128
DEG_G = NPAD // 128

_MESH = plsc.VectorSubcoreMesh(core_axis_name="c", subcore_axis_name="s")


@functools.partial(
    pl.kernel,
    out_type=jax.ShapeDtypeStruct((NC, NPAD, HID), jnp.float32),
    mesh=_MESH,
    scratch_types=[
        pltpu.VMEM((E_PER_W,), jnp.int32),
        pltpu.VMEM((CHUNK, HID), jnp.float32),
        pltpu.VMEM((STRIPE, HID), jnp.float32),
        pltpu.VMEM_SHARED((NPAD, HID), jnp.float32),
        pltpu.SemaphoreType.DMA,
    ],
    compiler_params=pltpu.CompilerParams(use_tc_tiling_on_sc=False),
)
def _deg_kernel(edge_hbm, out_hbm, dst_v, ones_v, zero_v, hist_sh, sem):
    c = lax.axis_index("c")
    s = lax.axis_index("s")
    wid = c * NS + s

    def zb(i, _):
        zero_v[i] = jnp.zeros((HID,), jnp.float32)
        return 0
    lax.fori_loop(0, STRIPE, zb, 0, unroll=8)

    def ob(i, _):
        ones_v[i] = jnp.ones((HID,), jnp.float32)
        return 0
    lax.fori_loop(0, CHUNK, ob, 0, unroll=8)

    pltpu.sync_copy(zero_v, hist_sh.at[pl.ds(s * STRIPE, STRIPE)])
    plsc.subcore_barrier()

    pltpu.sync_copy(edge_hbm.at[1, pl.ds(wid * E_PER_W, E_PER_W)], dst_v)

    def body(j, _):
        pltpu.async_copy(
            ones_v, hist_sh.at[dst_v.at[pl.ds(j * CHUNK, CHUNK)]], sem,
            add=True)
        return 0
    lax.fori_loop(0, NCHUNK, body, 0)

    def drain(j, _):
        pltpu.make_async_copy(
            ones_v, hist_sh.at[dst_v.at[pl.ds(0, CHUNK)]], sem).wait()
        return 0
    lax.fori_loop(0, NCHUNK, drain, 0)

    plsc.subcore_barrier()
    pltpu.sync_copy(hist_sh.at[pl.ds(s * STRIPE, STRIPE)],
                    out_hbm.at[c, pl.ds(s * STRIPE, STRIPE)])


NSLOT = 5
NOUTER = NCHUNK // NSLOT


@functools.partial(
    pl.kernel,
    out_type=jax.ShapeDtypeStruct((NC, NPAD, HID), jnp.float32),
    mesh=_MESH,
    scratch_types=[
        pltpu.VMEM((E_PER_W,), jnp.int32),
        pltpu.VMEM((E_PER_W,), jnp.int32),
        [pltpu.VMEM((CHUNK, HID), jnp.float32) for _ in range(NSLOT)],
        pltpu.VMEM((STRIPE, HID), jnp.float32),
        pltpu.VMEM_SHARED((NPAD, HID), jnp.float32),
        pltpu.VMEM_SHARED((N_NODES, HID), jnp.float32),
        [pltpu.SemaphoreType.DMA for _ in range(NSLOT)],
        [pltpu.SemaphoreType.DMA for _ in range(NSLOT)],
    ],
    compiler_params=pltpu.CompilerParams(use_tc_tiling_on_sc=False),
)
def _mp_kernel(xs_hbm, edge_hbm, out_hbm,
               src_v, dst_v, bufs, zero_v, acc_sh, xs_sh, gsems, ssems):
    c = lax.axis_index("c")
    s = lax.axis_index("s")
    wid = c * NS + s
    XSTRIPE = N_NODES // NS

    def zbody(i, _):
        zero_v[i] = jnp.zeros((HID,), jnp.float32)
        return 0
    lax.fori_loop(0, STRIPE, zbody, 0, unroll=8)
    pltpu.sync_copy(zero_v, acc_sh.at[pl.ds(s * STRIPE, STRIPE)])
    pltpu.sync_copy(xs_hbm.at[pl.ds(s * XSTRIPE, XSTRIPE)],
                    xs_sh.at[pl.ds(s * XSTRIPE, XSTRIPE)])
    plsc.subcore_barrier()

    pltpu.sync_copy(edge_hbm.at[0, pl.ds(wid * E_PER_W, E_PER_W)], src_v)
    pltpu.sync_copy(edge_hbm.at[1, pl.ds(wid * E_PER_W, E_PER_W)], dst_v)

    def sidx(j):
        return src_v.at[pl.ds(j * CHUNK, CHUNK)]

    def didx(j):
        return dst_v.at[pl.ds(j * CHUNK, CHUNK)]

    pltpu.async_copy(xs_sh.at[sidx(0)], bufs[0], gsems[0])
    pltpu.async_copy(xs_sh.at[sidx(1)], bufs[1], gsems[1])

    def body(t, _):
        for b in range(NSLOT):
            j = t * NSLOT + b
            p2 = (b + 2) % NSLOT

            @pl.when(j >= 3)
            def _():
                pltpu.make_async_copy(
                    bufs[p2], acc_sh.at[didx(0)], ssems[p2]).wait()

            @pl.when(j + 2 < NCHUNK)
            def _():
                pltpu.async_copy(xs_sh.at[sidx(j + 2)], bufs[p2],
                                 gsems[p2])

            pltpu.make_async_copy(xs_sh.at[sidx(j)], bufs[b],
                                  gsems[b]).wait()
            pltpu.async_copy(bufs[b], acc_sh.at[didx(j)], ssems[b],
                             add=True)
        return 0
    lax.fori_loop(0, NOUTER, body, 0)

    for b in (2, 3, 4):
        pltpu.make_async_copy(bufs[b], acc_sh.at[didx(0)], ssems[b]).wait()

    plsc.subcore_barrier()
    pltpu.sync_copy(acc_sh.at[pl.ds(s * STRIPE, STRIPE)],
                    out_hbm.at[c, pl.ds(s * STRIPE, STRIPE)])


def _tc_mm1_body(xr_ref, w1b_ref, xw_ref):
    xw_ref[...] = jnp.dot(xr_ref[...], w1b_ref[...],
                          preferred_element_type=jnp.float32)


def _tc_scale_body(xw_ref, degp_ref, xs1_ref, dis_ref):
    deg = degp_ref[0, :GRP] + degp_ref[1, :GRP] + 1.0
    dis_g = lax.rsqrt(deg)
    dis_ref[...] = dis_g
    xs1_ref[...] = dis_g * xw_ref[...]


def _tc_mid_body(accp_ref, xs1_ref, dis_ref, w2b_ref, b1t_ref, xs2_ref):
    acc = accp_ref[0, :GRP] + accp_ref[1, :GRP] + xs1_ref[...]
    dis_g = dis_ref[...]
    h1 = dis_g * acc + b1t_ref[...]
    h1 = jnp.where(h1 > 0, h1, jnp.exp(jnp.minimum(h1, 0.0)) - 1.0)
    xw2 = jnp.dot(h1, w2b_ref[...], preferred_element_type=jnp.float32)
    xs2_ref[...] = dis_g * xw2


def _tc_final_body(accp_ref, xs2_ref, dis_ref, wlb_ref, b2t_ref, blr_ref,
                   out_ref):
    acc = accp_ref[0, :GRP] + accp_ref[1, :GRP] + xs2_ref[...]
    h2 = dis_ref[...] * acc + b2t_ref[...]
    og = jnp.dot(h2, wlb_ref[...], preferred_element_type=jnp.float32)
    out_ref[...] = og + blr_ref[...]


def kernel(x, edge_index, W1, b1, W2, b2, Wl, bl):
    e32 = edge_index.astype(jnp.int32)

    eye8 = jnp.eye(8, dtype=jnp.float32)
    w1b = jnp.kron(eye8, W1)
    w2b = jnp.kron(eye8, W2)
    wlb = jnp.kron(eye8, Wl)
    b1t = jnp.tile(b1, 8).reshape(1, 128)
    b2t = jnp.tile(b2, 8).reshape(1, 128)
    blr = bl.reshape(1, 1)
    xr = x.reshape(GRP, 8 * IN_CH)

    degp = _deg_kernel(e32)
    degp_g = degp.reshape(NC, GRP_PAD, 128)

    xw1_g = pl.pallas_call(
        _tc_mm1_body,
        out_shape=jax.ShapeDtypeStruct((GRP, 128), jnp.float32),
    )(xr, w1b)

    xs1_g, dis_g = pl.pallas_call(
        _tc_scale_body,
        out_shape=(
            jax.ShapeDtypeStruct((GRP, 128), jnp.float32),
            jax.ShapeDtypeStruct((GRP, 128), jnp.float32),
        ),
    )(xw1_g, degp_g)

    acc1 = _mp_kernel(xs1_g.reshape(N_NODES, HID), e32)
    acc1_g = acc1.reshape(NC, GRP_PAD, 128)

    xs2_g = pl.pallas_call(
        _tc_mid_body,
        out_shape=jax.ShapeDtypeStruct((GRP, 128), jnp.float32),
    )(acc1_g, xs1_g, dis_g, w2b, b1t)

    acc2 = _mp_kernel(xs2_g.reshape(N_NODES, HID), e32)
    acc2_g = acc2.reshape(NC, GRP_PAD, 128)

    out_g = pl.pallas_call(
        _tc_final_body,
        out_shape=jax.ShapeDtypeStruct((GRP, 8), jnp.float32),
    )(acc2_g, xs2_g, dis_g, wlb, b2t, blr)

    return out_g.reshape(N_NODES, 1)
